# baseline (speedup 1.0000x reference)
import functools

import jax
import jax.numpy as jnp
from jax import lax
from jax.experimental import pallas as pl
from jax.experimental.pallas import tpu as pltpu
from jax.experimental.pallas import tpu_sc as plsc

N = 10000
D = 128
E = 320000

NC = 2
NS = 16
LANES = 16
CHUNK = 128
NCHUNK = 80
EPT = CHUNK * NCHUNK
EPAD = NC * NS * EPT

NPAD = 10240
RPT = NPAD // NS

MPAD = 10112
MRPT = MPAD // NS

_SC_MESH = plsc.VectorSubcoreMesh(core_axis_name="c", subcore_axis_name="s")


@functools.partial(
    pl.kernel,
    out_type=jax.ShapeDtypeStruct((NC, NPAD), jnp.float32),
    mesh=_SC_MESH,
    scratch_types=[
        pltpu.VMEM((NCHUNK, CHUNK), jnp.int32),
        pltpu.VMEM((NPAD,), jnp.float32),
        pltpu.VMEM((NS, RPT), jnp.float32),
        pltpu.VMEM((RPT,), jnp.float32),
        pltpu.VMEM_SHARED((NS, NPAD), jnp.float32),
    ],
    compiler_params=pltpu.CompilerParams(needs_layout_passes=False),
)
def _deg_kernel(col_hbm, out_hbm, col_v, hist_v, red_v, out_v, hists_sh):
    c = lax.axis_index("c")
    s = lax.axis_index("s")

    def _zfill(i, carry):
        hist_v[pl.ds(i * LANES, LANES)] = jnp.zeros((LANES,), jnp.float32)
        return carry

    lax.fori_loop(0, NPAD // LANES, _zfill, 0)

    pltpu.sync_copy(col_hbm.at[c, s], col_v)

    def _vreg(i, carry):
        j = i // (CHUNK // LANES)
        k = i % (CHUNK // LANES)
        idx = col_v[j, pl.ds(k * LANES, LANES)]
        counts, last = plsc.scan_count(idx)
        plsc.addupdate_scatter(hist_v, [idx], counts.astype(jnp.float32),
                               mask=last)
        return carry

    lax.fori_loop(0, EPT // LANES, _vreg, 0)

    pltpu.sync_copy(hist_v, hists_sh.at[s])
    plsc.subcore_barrier()
    pltpu.sync_copy(hists_sh.at[:, pl.ds(s * RPT, RPT)], red_v)

    def _red(v, carry):
        acc = jnp.zeros((LANES,), jnp.float32)
        for t in range(NS):
            acc = acc + red_v[t, pl.ds(v * LANES, LANES)]
        out_v[pl.ds(v * LANES, LANES)] = acc
        return carry

    lax.fori_loop(0, RPT // LANES, _red, 0)
    pltpu.sync_copy(out_v, out_hbm.at[c, pl.ds(s * RPT, RPT)])


MCHUNK = 128
MTOT = EPAD // (NS * MCHUNK)
MK0 = MTOT // NC
HBLK = 64
GSPLIT = 2


@functools.partial(
    pl.kernel,
    out_type=jax.ShapeDtypeStruct((NC, MPAD, D), jnp.float32),
    mesh=_SC_MESH,
    scratch_types=[
        pltpu.VMEM((HBLK, MCHUNK), jnp.int32),
        pltpu.VMEM((HBLK, MCHUNK), jnp.int32),
        pltpu.VMEM((2, MCHUNK, D), jnp.float32),
        pltpu.VMEM_SHARED((MPAD, D), jnp.float32),
        pltpu.SemaphoreType.DMA((2, GSPLIT)),
    ],
)
def _msg_kernel(y_hbm, row_hbm, col_hbm, out_hbm,
                row_v, col_v, rows_v, acc_sh, sem):
    c = lax.axis_index("c")
    s = lax.axis_index("s")

    def _zfill(i, carry):
        for jj in range(D // LANES):
            rows_v[0, i, pl.ds(jj * LANES, LANES)] = jnp.zeros((LANES,),
                                                               jnp.float32)
        return carry

    lax.fori_loop(0, MCHUNK, _zfill, 0)
    for k in range(MRPT // MCHUNK):
        pltpu.sync_copy(rows_v.at[0],
                        acc_sh.at[pl.ds(s * MRPT + k * MCHUNK, MCHUNK)])
    _ztail = MRPT % MCHUNK
    if _ztail:
        pltpu.sync_copy(rows_v.at[0, pl.ds(0, _ztail)],
                        acc_sh.at[pl.ds(s * MRPT + MRPT - _ztail, _ztail)])
    plsc.subcore_barrier()

    GS = MCHUNK // GSPLIT

    def _gather(j, b):
        for g in range(GSPLIT):
            pltpu.async_copy(y_hbm.at[row_v.at[j, pl.ds(g * GS, GS)]],
                             rows_v.at[b, pl.ds(g * GS, GS)], sem.at[b, g])

    def _gather_wait(j, b):
        for g in range(GSPLIT):
            pltpu.make_async_copy(y_hbm.at[row_v.at[j, pl.ds(g * GS, GS)]],
                                  rows_v.at[b, pl.ds(g * GS, GS)],
                                  sem.at[b, g]).wait()

    def _block(base, nch):
        pltpu.sync_copy(row_hbm.at[s, pl.ds(base, nch)], row_v.at[pl.ds(0, nch)])
        pltpu.sync_copy(col_hbm.at[s, pl.ds(base, nch)], col_v.at[pl.ds(0, nch)])
        _gather(0, 0)

        def _chunk(j, carry):
            b = lax.rem(j, 2)

            @pl.when(j + 1 < nch)
            def _():
                _gather(j + 1, 1 - b)

            _gather_wait(j, b)
            pltpu.sync_copy(rows_v.at[b], acc_sh.at[col_v.at[j]], add=True)
            return carry

        lax.fori_loop(0, nch, _chunk, 0)

    for bs in range(MK0 // HBLK):
        _block(c * MK0 + bs * HBLK, HBLK)
    _btail = MK0 % HBLK
    if _btail:
        _block(c * MK0 + MK0 - _btail, _btail)

    plsc.subcore_barrier()

    pltpu.sync_copy(acc_sh.at[pl.ds(s * MRPT, MRPT)],
                    out_hbm.at[c, pl.ds(s * MRPT, MRPT)])


def _tc1_body(x_ref, w_ref, degp_ref, y_ref, dinv_ref):
    deg = (degp_ref[0] + degp_ref[1])[:N] + 2.0
    dinv = lax.rsqrt(deg)
    xw = jnp.dot(x_ref[...], w_ref[...], preferred_element_type=jnp.float32)
    y_ref[...] = xw * dinv
    dinv_ref[...] = dinv


def _tc2_body(acc_ref, y_ref, dinv_ref, b_ref, w_ref, y2_ref):
    dinv = dinv_ref[...]
    acc = (acc_ref[0] + acc_ref[1])[:N]
    h = (acc + 2.0 * y_ref[...]) * dinv + b_ref[...][None, :]
    h = jnp.maximum(h, 0.0)
    y2_ref[...] = jnp.dot(h, w_ref[...], preferred_element_type=jnp.float32) * dinv


def _tc3_body(acc_ref, y2_ref, dinv_ref, b_ref, out_ref):
    acc = (acc_ref[0] + acc_ref[1])[:N]
    h = (acc + 2.0 * y2_ref[...]) * dinv_ref[...] \
        + b_ref[...][None, :]
    out_ref[...] = jnp.maximum(h, 0.0)


_tc1 = pl.pallas_call(
    _tc1_body,
    out_shape=(jax.ShapeDtypeStruct((N, D), jnp.float32),
               jax.ShapeDtypeStruct((N, 1), jnp.float32)),
)

_tc2 = pl.pallas_call(
    _tc2_body,
    out_shape=jax.ShapeDtypeStruct((N, D), jnp.float32),
)

_tc3 = pl.pallas_call(
    _tc3_body,
    out_shape=jax.ShapeDtypeStruct((N, D), jnp.float32),
)


def kernel(x, edge_index, W1, b1, W2, b2):
    pad = EPAD - E
    pidx = jnp.arange(pad, dtype=jnp.int32)
    row_flat = jnp.concatenate([edge_index[0], pidx % CHUNK])
    col_flat = jnp.concatenate([edge_index[1], N + pidx % (MPAD - N)])
    row_m = row_flat.reshape(NS, MTOT, MCHUNK)
    col_m = col_flat.reshape(NS, MTOT, MCHUNK)
    col_d = col_flat.reshape(NC, NS, NCHUNK, CHUNK)

    degp = _deg_kernel(col_d).reshape(NC, NPAD, 1)
    y1, dinv = _tc1(x, W1, degp)
    acc1 = _msg_kernel(y1, row_m, col_m)
    y2 = _tc2(acc1, y1, dinv, b1, W2)
    acc2 = _msg_kernel(y2, row_m, col_m)
    return _tc3(acc2, y2, dinv, b2)


# speedup vs baseline: 1.0263x; 1.0263x over previous
import functools

import jax
import jax.numpy as jnp
from jax import lax
from jax.experimental import pallas as pl
from jax.experimental.pallas import tpu as pltpu
from jax.experimental.pallas import tpu_sc as plsc

N = 10000
D = 128
E = 320000

NC = 2
NS = 16
LANES = 16
CHUNK = 128
NCHUNK = 80
EPT = CHUNK * NCHUNK
EPAD = NC * NS * EPT

NPAD = 10240
RPT = NPAD // NS

MPAD = 10112
MRPT = MPAD // NS

_SC_MESH = plsc.VectorSubcoreMesh(core_axis_name="c", subcore_axis_name="s")


@functools.partial(
    pl.kernel,
    out_type=jax.ShapeDtypeStruct((NC, NPAD), jnp.float32),
    mesh=_SC_MESH,
    scratch_types=[
        pltpu.VMEM((NCHUNK, CHUNK), jnp.int32),
        pltpu.VMEM((NPAD,), jnp.float32),
        pltpu.VMEM((NS, RPT), jnp.float32),
        pltpu.VMEM((RPT,), jnp.float32),
        pltpu.VMEM_SHARED((NS, NPAD), jnp.float32),
    ],
    compiler_params=pltpu.CompilerParams(needs_layout_passes=False),
)
def _deg_kernel(col_hbm, out_hbm, col_v, hist_v, red_v, out_v, hists_sh):
    c = lax.axis_index("c")
    s = lax.axis_index("s")

    def _zfill(i, carry):
        hist_v[pl.ds(i * LANES, LANES)] = jnp.zeros((LANES,), jnp.float32)
        return carry

    lax.fori_loop(0, NPAD // LANES, _zfill, 0)

    pltpu.sync_copy(col_hbm.at[c, s], col_v)

    def _vreg(i, carry):
        j = i // (CHUNK // LANES)
        k = i % (CHUNK // LANES)
        idx = col_v[j, pl.ds(k * LANES, LANES)]
        counts, last = plsc.scan_count(idx)
        plsc.addupdate_scatter(hist_v, [idx], counts.astype(jnp.float32),
                               mask=last)
        return carry

    lax.fori_loop(0, EPT // LANES, _vreg, 0)

    pltpu.sync_copy(hist_v, hists_sh.at[s])
    plsc.subcore_barrier()
    pltpu.sync_copy(hists_sh.at[:, pl.ds(s * RPT, RPT)], red_v)

    def _red(v, carry):
        acc = jnp.zeros((LANES,), jnp.float32)
        for t in range(NS):
            acc = acc + red_v[t, pl.ds(v * LANES, LANES)]
        out_v[pl.ds(v * LANES, LANES)] = acc
        return carry

    lax.fori_loop(0, RPT // LANES, _red, 0)
    pltpu.sync_copy(out_v, out_hbm.at[c, pl.ds(s * RPT, RPT)])


MCHUNK = 128
MTOT = EPAD // (NS * MCHUNK)
MK0 = MTOT // NC
HBLK = 64
GSPLIT = 1


@functools.partial(
    pl.kernel,
    out_type=jax.ShapeDtypeStruct((NC, MPAD, D), jnp.float32),
    mesh=_SC_MESH,
    scratch_types=[
        pltpu.VMEM((HBLK, MCHUNK), jnp.int32),
        pltpu.VMEM((HBLK, MCHUNK), jnp.int32),
        pltpu.VMEM((2, MCHUNK, D), jnp.float32),
        pltpu.VMEM_SHARED((MPAD, D), jnp.float32),
        pltpu.SemaphoreType.DMA((2, GSPLIT)),
    ],
)
def _msg_kernel(y_hbm, row_hbm, col_hbm, out_hbm,
                row_v, col_v, rows_v, acc_sh, sem):
    c = lax.axis_index("c")
    s = lax.axis_index("s")

    def _zfill(i, carry):
        for jj in range(D // LANES):
            rows_v[0, i, pl.ds(jj * LANES, LANES)] = jnp.zeros((LANES,),
                                                               jnp.float32)
        return carry

    lax.fori_loop(0, MCHUNK, _zfill, 0)
    for k in range(MRPT // MCHUNK):
        pltpu.sync_copy(rows_v.at[0],
                        acc_sh.at[pl.ds(s * MRPT + k * MCHUNK, MCHUNK)])
    _ztail = MRPT % MCHUNK
    if _ztail:
        pltpu.sync_copy(rows_v.at[0, pl.ds(0, _ztail)],
                        acc_sh.at[pl.ds(s * MRPT + MRPT - _ztail, _ztail)])
    plsc.subcore_barrier()

    GS = MCHUNK // GSPLIT

    def _gather(j, b):
        for g in range(GSPLIT):
            pltpu.async_copy(y_hbm.at[row_v.at[j, pl.ds(g * GS, GS)]],
                             rows_v.at[b, pl.ds(g * GS, GS)], sem.at[b, g])

    def _gather_wait(j, b):
        for g in range(GSPLIT):
            pltpu.make_async_copy(y_hbm.at[row_v.at[j, pl.ds(g * GS, GS)]],
                                  rows_v.at[b, pl.ds(g * GS, GS)],
                                  sem.at[b, g]).wait()

    def _block(base, nch):
        pltpu.sync_copy(row_hbm.at[s, pl.ds(base, nch)], row_v.at[pl.ds(0, nch)])
        pltpu.sync_copy(col_hbm.at[s, pl.ds(base, nch)], col_v.at[pl.ds(0, nch)])
        _gather(0, 0)

        def _chunk(j, carry):
            b = lax.rem(j, 2)

            @pl.when(j + 1 < nch)
            def _():
                _gather(j + 1, 1 - b)

            _gather_wait(j, b)
            pltpu.sync_copy(rows_v.at[b], acc_sh.at[col_v.at[j]], add=True)
            return carry

        lax.fori_loop(0, nch, _chunk, 0)

    for bs in range(MK0 // HBLK):
        _block(c * MK0 + bs * HBLK, HBLK)
    _btail = MK0 % HBLK
    if _btail:
        _block(c * MK0 + MK0 - _btail, _btail)

    plsc.subcore_barrier()

    pltpu.sync_copy(acc_sh.at[pl.ds(s * MRPT, MRPT)],
                    out_hbm.at[c, pl.ds(s * MRPT, MRPT)])


BN = 1024
GRID = NPAD // BN


def _tc1_body(x_ref, w_ref, degp_ref, y_ref, dinv_ref):
    deg = degp_ref[0] + degp_ref[1] + 2.0
    dinv = lax.rsqrt(deg).reshape(BN, 1)
    xw = jnp.dot(x_ref[...], w_ref[...], preferred_element_type=jnp.float32)
    y_ref[...] = xw * dinv
    dinv_ref[...] = dinv


def _tc2_body(acc_ref, y_ref, dinv_ref, b_ref, w_ref, y2_ref):
    dinv = dinv_ref[...]
    h = (acc_ref[0] + acc_ref[1] + 2.0 * y_ref[...]) * dinv + b_ref[...]
    h = jnp.maximum(h, 0.0)
    y2_ref[...] = jnp.dot(h, w_ref[...], preferred_element_type=jnp.float32) * dinv


def _tc3_body(acc_ref, y2_ref, dinv_ref, b_ref, out_ref):
    h = (acc_ref[0] + acc_ref[1] + 2.0 * y2_ref[...]) * dinv_ref[...] + b_ref[...]
    out_ref[...] = jnp.maximum(h, 0.0)


_tc1 = pl.pallas_call(
    _tc1_body,
    grid=(GRID,),
    in_specs=[
        pl.BlockSpec((BN, D), lambda i: (i, 0)),
        pl.BlockSpec((D, D), lambda i: (0, 0)),
        pl.BlockSpec((NC, BN), lambda i: (0, i)),
    ],
    out_specs=(pl.BlockSpec((BN, D), lambda i: (i, 0)),
               pl.BlockSpec((BN, 1), lambda i: (i, 0))),
    out_shape=(jax.ShapeDtypeStruct((N, D), jnp.float32),
               jax.ShapeDtypeStruct((N, 1), jnp.float32)),
)

_tc2 = pl.pallas_call(
    _tc2_body,
    grid=(GRID,),
    in_specs=[
        pl.BlockSpec((NC, BN, D), lambda i: (0, i, 0)),
        pl.BlockSpec((BN, D), lambda i: (i, 0)),
        pl.BlockSpec((BN, 1), lambda i: (i, 0)),
        pl.BlockSpec((1, D), lambda i: (0, 0)),
        pl.BlockSpec((D, D), lambda i: (0, 0)),
    ],
    out_specs=pl.BlockSpec((BN, D), lambda i: (i, 0)),
    out_shape=jax.ShapeDtypeStruct((N, D), jnp.float32),
)

_tc3 = pl.pallas_call(
    _tc3_body,
    grid=(GRID,),
    in_specs=[
        pl.BlockSpec((NC, BN, D), lambda i: (0, i, 0)),
        pl.BlockSpec((BN, D), lambda i: (i, 0)),
        pl.BlockSpec((BN, 1), lambda i: (i, 0)),
        pl.BlockSpec((1, D), lambda i: (0, 0)),
    ],
    out_specs=pl.BlockSpec((BN, D), lambda i: (i, 0)),
    out_shape=jax.ShapeDtypeStruct((N, D), jnp.float32),
)


def kernel(x, edge_index, W1, b1, W2, b2):
    pad = EPAD - E
    pidx = jnp.arange(pad, dtype=jnp.int32)
    row_flat = jnp.concatenate([edge_index[0], pidx % MCHUNK])
    col_flat = jnp.concatenate([edge_index[1], N + pidx % (MPAD - N)])
    row_m = row_flat.reshape(NS, MTOT, MCHUNK)
    col_m = col_flat.reshape(NS, MTOT, MCHUNK)
    col_d = col_flat.reshape(NC, NS, NCHUNK, CHUNK)
    b1r = b1.reshape(1, D)
    b2r = b2.reshape(1, D)

    degp = _deg_kernel(col_d)
    y1, dinv = _tc1(x, W1, degp)
    acc1 = _msg_kernel(y1, row_m, col_m)
    y2 = _tc2(acc1, y1, dinv, b1r, W2)
    acc2 = _msg_kernel(y2, row_m, col_m)
    return _tc3(acc2, y2, dinv, b2r)


# speedup vs baseline: 1.0288x; 1.0025x over previous
import functools

import jax
import jax.numpy as jnp
from jax import lax
from jax.experimental import pallas as pl
from jax.experimental.pallas import tpu as pltpu
from jax.experimental.pallas import tpu_sc as plsc

N = 10000
D = 128
E = 320000

NC = 2
NS = 16
LANES = 16
CHUNK = 128
NCHUNK = 80
EPT = CHUNK * NCHUNK
EPAD = NC * NS * EPT

NPAD = 10240
RPT = NPAD // NS

MPAD = 10112
MRPT = MPAD // NS

_SC_MESH = plsc.VectorSubcoreMesh(core_axis_name="c", subcore_axis_name="s")


@functools.partial(
    pl.kernel,
    out_type=jax.ShapeDtypeStruct((NC, NPAD), jnp.float32),
    mesh=_SC_MESH,
    scratch_types=[
        pltpu.VMEM((NCHUNK, CHUNK), jnp.int32),
        pltpu.VMEM((NPAD,), jnp.float32),
        pltpu.VMEM((NS, RPT), jnp.float32),
        pltpu.VMEM((RPT,), jnp.float32),
        pltpu.VMEM_SHARED((NS, NPAD), jnp.float32),
    ],
    compiler_params=pltpu.CompilerParams(needs_layout_passes=False),
)
def _deg_kernel(col_hbm, out_hbm, col_v, hist_v, red_v, out_v, hists_sh):
    c = lax.axis_index("c")
    s = lax.axis_index("s")

    def _zfill(i, carry):
        hist_v[pl.ds(i * LANES, LANES)] = jnp.zeros((LANES,), jnp.float32)
        return carry

    lax.fori_loop(0, NPAD // LANES, _zfill, 0)

    pltpu.sync_copy(col_hbm.at[c, s], col_v)

    def _vreg(i, carry):
        j = i // (CHUNK // LANES)
        k = i % (CHUNK // LANES)
        idx = col_v[j, pl.ds(k * LANES, LANES)]
        counts, last = plsc.scan_count(idx)
        plsc.addupdate_scatter(hist_v, [idx], counts.astype(jnp.float32),
                               mask=last)
        return carry

    lax.fori_loop(0, EPT // LANES, _vreg, 0)

    pltpu.sync_copy(hist_v, hists_sh.at[s])
    plsc.subcore_barrier()
    pltpu.sync_copy(hists_sh.at[:, pl.ds(s * RPT, RPT)], red_v)

    def _red(v, carry):
        acc = jnp.zeros((LANES,), jnp.float32)
        for t in range(NS):
            acc = acc + red_v[t, pl.ds(v * LANES, LANES)]
        out_v[pl.ds(v * LANES, LANES)] = acc
        return carry

    lax.fori_loop(0, RPT // LANES, _red, 0)
    pltpu.sync_copy(out_v, out_hbm.at[c, pl.ds(s * RPT, RPT)])


MCHUNK = 128
MTOT = EPAD // (NS * MCHUNK)
MK0 = MTOT // NC
HBLK = 64
GSPLIT = 1


@functools.partial(
    pl.kernel,
    out_type=jax.ShapeDtypeStruct((NC, MPAD, D), jnp.float32),
    mesh=_SC_MESH,
    scratch_types=[
        pltpu.VMEM((HBLK, MCHUNK), jnp.int32),
        pltpu.VMEM((HBLK, MCHUNK), jnp.int32),
        pltpu.VMEM((2, MCHUNK, D), jnp.float32),
        pltpu.VMEM_SHARED((MPAD, D), jnp.float32),
        pltpu.SemaphoreType.DMA((2, GSPLIT)),
    ],
)
def _msg_kernel(y_hbm, row_hbm, col_hbm, out_hbm,
                row_v, col_v, rows_v, acc_sh, sem):
    c = lax.axis_index("c")
    s = lax.axis_index("s")

    def _zfill(i, carry):
        for jj in range(D // LANES):
            rows_v[0, i, pl.ds(jj * LANES, LANES)] = jnp.zeros((LANES,),
                                                               jnp.float32)
        return carry

    lax.fori_loop(0, MCHUNK, _zfill, 0)
    for k in range(MRPT // MCHUNK):
        pltpu.sync_copy(rows_v.at[0],
                        acc_sh.at[pl.ds(s * MRPT + k * MCHUNK, MCHUNK)])
    _ztail = MRPT % MCHUNK
    if _ztail:
        pltpu.sync_copy(rows_v.at[0, pl.ds(0, _ztail)],
                        acc_sh.at[pl.ds(s * MRPT + MRPT - _ztail, _ztail)])
    plsc.subcore_barrier()

    GS = MCHUNK // GSPLIT

    def _gather(j, b):
        for g in range(GSPLIT):
            pltpu.async_copy(y_hbm.at[row_v.at[j, pl.ds(g * GS, GS)]],
                             rows_v.at[b, pl.ds(g * GS, GS)], sem.at[b, g])

    def _gather_wait(j, b):
        for g in range(GSPLIT):
            pltpu.make_async_copy(y_hbm.at[row_v.at[j, pl.ds(g * GS, GS)]],
                                  rows_v.at[b, pl.ds(g * GS, GS)],
                                  sem.at[b, g]).wait()

    def _block(base, nch):
        pltpu.sync_copy(row_hbm.at[s, pl.ds(base, nch)], row_v.at[pl.ds(0, nch)])
        pltpu.sync_copy(col_hbm.at[s, pl.ds(base, nch)], col_v.at[pl.ds(0, nch)])
        _gather(0, 0)

        def _chunk(j, carry):
            b = lax.rem(j, 2)

            @pl.when(j + 1 < nch)
            def _():
                _gather(j + 1, 1 - b)

            _gather_wait(j, b)
            pltpu.sync_copy(rows_v.at[b], acc_sh.at[col_v.at[j]], add=True)
            return carry

        lax.fori_loop(0, nch, _chunk, 0)

    for bs in range(MK0 // HBLK):
        _block(c * MK0 + bs * HBLK, HBLK)
    _btail = MK0 % HBLK
    if _btail:
        _block(c * MK0 + MK0 - _btail, _btail)

    plsc.subcore_barrier()

    pltpu.sync_copy(acc_sh.at[pl.ds(s * MRPT, MRPT)],
                    out_hbm.at[c, pl.ds(s * MRPT, MRPT)])


BN = 1024
GRID = NPAD // BN


def _tcmm_body(x_ref, w_ref, xw_ref):
    xw_ref[...] = jnp.dot(x_ref[...], w_ref[...],
                          preferred_element_type=jnp.float32)


def _tc1_body(xw_ref, degp_ref, y_ref, dinv_ref):
    deg = degp_ref[0] + degp_ref[1] + 2.0
    dinv = lax.rsqrt(deg).reshape(BN, 1)
    y_ref[...] = xw_ref[...] * dinv
    dinv_ref[...] = dinv


def _tc2_body(acc_ref, y_ref, dinv_ref, b_ref, w_ref, y2_ref):
    dinv = dinv_ref[...]
    h = (acc_ref[0] + acc_ref[1] + 2.0 * y_ref[...]) * dinv + b_ref[...]
    h = jnp.maximum(h, 0.0)
    y2_ref[...] = jnp.dot(h, w_ref[...], preferred_element_type=jnp.float32) * dinv


def _tc3_body(acc_ref, y2_ref, dinv_ref, b_ref, out_ref):
    h = (acc_ref[0] + acc_ref[1] + 2.0 * y2_ref[...]) * dinv_ref[...] + b_ref[...]
    out_ref[...] = jnp.maximum(h, 0.0)


_tcmm = pl.pallas_call(
    _tcmm_body,
    grid=(GRID,),
    in_specs=[
        pl.BlockSpec((BN, D), lambda i: (i, 0)),
        pl.BlockSpec((D, D), lambda i: (0, 0)),
    ],
    out_specs=pl.BlockSpec((BN, D), lambda i: (i, 0)),
    out_shape=jax.ShapeDtypeStruct((N, D), jnp.float32),
)

_tc1 = pl.pallas_call(
    _tc1_body,
    grid=(GRID,),
    in_specs=[
        pl.BlockSpec((BN, D), lambda i: (i, 0)),
        pl.BlockSpec((NC, BN), lambda i: (0, i)),
    ],
    out_specs=(pl.BlockSpec((BN, D), lambda i: (i, 0)),
               pl.BlockSpec((BN, 1), lambda i: (i, 0))),
    out_shape=(jax.ShapeDtypeStruct((N, D), jnp.float32),
               jax.ShapeDtypeStruct((N, 1), jnp.float32)),
)

_tc2 = pl.pallas_call(
    _tc2_body,
    grid=(GRID,),
    in_specs=[
        pl.BlockSpec((NC, BN, D), lambda i: (0, i, 0)),
        pl.BlockSpec((BN, D), lambda i: (i, 0)),
        pl.BlockSpec((BN, 1), lambda i: (i, 0)),
        pl.BlockSpec((1, D), lambda i: (0, 0)),
        pl.BlockSpec((D, D), lambda i: (0, 0)),
    ],
    out_specs=pl.BlockSpec((BN, D), lambda i: (i, 0)),
    out_shape=jax.ShapeDtypeStruct((N, D), jnp.float32),
)

_tc3 = pl.pallas_call(
    _tc3_body,
    grid=(GRID,),
    in_specs=[
        pl.BlockSpec((NC, BN, D), lambda i: (0, i, 0)),
        pl.BlockSpec((BN, D), lambda i: (i, 0)),
        pl.BlockSpec((BN, 1), lambda i: (i, 0)),
        pl.BlockSpec((1, D), lambda i: (0, 0)),
    ],
    out_specs=pl.BlockSpec((BN, D), lambda i: (i, 0)),
    out_shape=jax.ShapeDtypeStruct((N, D), jnp.float32),
)


def kernel(x, edge_index, W1, b1, W2, b2):
    pad = EPAD - E
    pidx = jnp.arange(pad, dtype=jnp.int32)
    row_flat = jnp.concatenate([edge_index[0], pidx % MCHUNK])
    col_flat = jnp.concatenate([edge_index[1], N + pidx % (MPAD - N)])
    row_m = row_flat.reshape(NS, MTOT, MCHUNK)
    col_m = col_flat.reshape(NS, MTOT, MCHUNK)
    col_d = col_flat.reshape(NC, NS, NCHUNK, CHUNK)
    b1r = b1.reshape(1, D)
    b2r = b2.reshape(1, D)

    degp = _deg_kernel(col_d)
    xw1 = _tcmm(x, W1)
    y1, dinv = _tc1(xw1, degp)
    acc1 = _msg_kernel(y1, row_m, col_m)
    y2 = _tc2(acc1, y1, dinv, b1r, W2)
    acc2 = _msg_kernel(y2, row_m, col_m)
    return _tc3(acc2, y2, dinv, b2r)


# speedup vs baseline: 1.0442x; 1.0149x over previous
import functools

import jax
import jax.numpy as jnp
from jax import lax
from jax.experimental import pallas as pl
from jax.experimental.pallas import tpu as pltpu
from jax.experimental.pallas import tpu_sc as plsc

N = 10000
D = 128
E = 320000

NC = 2
NS = 16
LANES = 16
CHUNK = 128
NCHUNK = 80
EPT = CHUNK * NCHUNK
EPAD = NC * NS * EPT

NPAD = 10240
RPT = NPAD // NS

MPAD = 10112
MRPT = MPAD // NS

_SC_MESH = plsc.VectorSubcoreMesh(core_axis_name="c", subcore_axis_name="s")


@functools.partial(
    pl.kernel,
    out_type=jax.ShapeDtypeStruct((NC, NPAD), jnp.float32),
    mesh=_SC_MESH,
    scratch_types=[
        pltpu.VMEM((NCHUNK, CHUNK), jnp.int32),
        pltpu.VMEM((NPAD,), jnp.float32),
        pltpu.VMEM((NS, RPT), jnp.float32),
        pltpu.VMEM((RPT,), jnp.float32),
        pltpu.VMEM_SHARED((NS, NPAD), jnp.float32),
    ],
    compiler_params=pltpu.CompilerParams(needs_layout_passes=False),
)
def _deg_kernel(col_hbm, out_hbm, col_v, hist_v, red_v, out_v, hists_sh):
    c = lax.axis_index("c")
    s = lax.axis_index("s")

    def _zfill(i, carry):
        hist_v[pl.ds(i * LANES, LANES)] = jnp.zeros((LANES,), jnp.float32)
        return carry

    lax.fori_loop(0, NPAD // LANES, _zfill, 0)

    pltpu.sync_copy(col_hbm.at[c, s], col_v)

    def _vreg(i, carry):
        j = i // (CHUNK // LANES)
        k = i % (CHUNK // LANES)
        idx = col_v[j, pl.ds(k * LANES, LANES)]
        counts, last = plsc.scan_count(idx)
        plsc.addupdate_scatter(hist_v, [idx], counts.astype(jnp.float32),
                               mask=last)
        return carry

    lax.fori_loop(0, EPT // LANES, _vreg, 0)

    pltpu.sync_copy(hist_v, hists_sh.at[s])
    plsc.subcore_barrier()
    pltpu.sync_copy(hists_sh.at[:, pl.ds(s * RPT, RPT)], red_v)

    def _red(v, carry):
        acc = jnp.zeros((LANES,), jnp.float32)
        for t in range(NS):
            acc = acc + red_v[t, pl.ds(v * LANES, LANES)]
        out_v[pl.ds(v * LANES, LANES)] = acc
        return carry

    lax.fori_loop(0, RPT // LANES, _red, 0)
    pltpu.sync_copy(out_v, out_hbm.at[c, pl.ds(s * RPT, RPT)])


MCHUNK = 128
MTOT = EPAD // (NS * MCHUNK)
MK0 = MTOT // NC
HBLK = 64
GSPLIT = 1


@functools.partial(
    pl.kernel,
    out_type=jax.ShapeDtypeStruct((NC, MPAD, D), jnp.float32),
    mesh=_SC_MESH,
    scratch_types=[
        pltpu.VMEM((HBLK, MCHUNK), jnp.int32),
        pltpu.VMEM((HBLK, MCHUNK), jnp.int32),
        pltpu.VMEM((2, MCHUNK, D), jnp.float32),
        pltpu.VMEM_SHARED((MPAD, D), jnp.float32),
        pltpu.SemaphoreType.DMA((2, GSPLIT)),
    ],
)
def _msg_kernel(y_hbm, row_hbm, col_hbm, out_hbm,
                row_v, col_v, rows_v, acc_sh, sem):
    c = lax.axis_index("c")
    s = lax.axis_index("s")

    def _zfill(i, carry):
        for jj in range(D // LANES):
            rows_v[0, i, pl.ds(jj * LANES, LANES)] = jnp.zeros((LANES,),
                                                               jnp.float32)
        return carry

    lax.fori_loop(0, MCHUNK, _zfill, 0)
    for k in range(MRPT // MCHUNK):
        pltpu.sync_copy(rows_v.at[0],
                        acc_sh.at[pl.ds(s * MRPT + k * MCHUNK, MCHUNK)])
    _ztail = MRPT % MCHUNK
    if _ztail:
        pltpu.sync_copy(rows_v.at[0, pl.ds(0, _ztail)],
                        acc_sh.at[pl.ds(s * MRPT + MRPT - _ztail, _ztail)])
    plsc.subcore_barrier()

    GS = MCHUNK // GSPLIT

    def _gather(j, b):
        for g in range(GSPLIT):
            pltpu.async_copy(y_hbm.at[row_v.at[j, pl.ds(g * GS, GS)]],
                             rows_v.at[b, pl.ds(g * GS, GS)], sem.at[b, g])

    def _gather_wait(j, b):
        for g in range(GSPLIT):
            pltpu.make_async_copy(y_hbm.at[row_v.at[j, pl.ds(g * GS, GS)]],
                                  rows_v.at[b, pl.ds(g * GS, GS)],
                                  sem.at[b, g]).wait()

    def _block(base, nch):
        pltpu.sync_copy(row_hbm.at[s, pl.ds(base, nch)], row_v.at[pl.ds(0, nch)])
        pltpu.sync_copy(col_hbm.at[s, pl.ds(base, nch)], col_v.at[pl.ds(0, nch)])
        _gather(0, 0)

        def _chunk(j, carry):
            b = lax.rem(j, 2)

            @pl.when(j + 1 < nch)
            def _():
                _gather(j + 1, 1 - b)

            _gather_wait(j, b)
            pltpu.sync_copy(rows_v.at[b], acc_sh.at[col_v.at[j]], add=True)
            return carry

        lax.fori_loop(0, nch, _chunk, 0)

    for bs in range(MK0 // HBLK):
        _block(c * MK0 + bs * HBLK, HBLK)
    _btail = MK0 % HBLK
    if _btail:
        _block(c * MK0 + MK0 - _btail, _btail)

    plsc.subcore_barrier()

    pltpu.sync_copy(acc_sh.at[pl.ds(s * MRPT, MRPT)],
                    out_hbm.at[c, pl.ds(s * MRPT, MRPT)])


BN = 2048
GRID = NPAD // BN


def _tcmm_body(x_ref, w_ref, xw_ref):
    xw_ref[...] = jnp.dot(x_ref[...], w_ref[...],
                          preferred_element_type=jnp.float32)


def _tc1_body(xw_ref, degp_ref, y_ref, dinv_ref):
    deg = degp_ref[0] + degp_ref[1] + 2.0
    dinv = lax.rsqrt(deg).reshape(BN, 1)
    y_ref[...] = xw_ref[...] * dinv
    dinv_ref[...] = dinv


def _tc2_body(acc_ref, y_ref, dinv_ref, b_ref, w_ref, y2_ref):
    dinv = dinv_ref[...]
    h = (acc_ref[0] + acc_ref[1] + 2.0 * y_ref[...]) * dinv + b_ref[...]
    h = jnp.maximum(h, 0.0)
    y2_ref[...] = jnp.dot(h, w_ref[...], preferred_element_type=jnp.float32) * dinv


def _tc3_body(acc_ref, y2_ref, dinv_ref, b_ref, out_ref):
    h = (acc_ref[0] + acc_ref[1] + 2.0 * y2_ref[...]) * dinv_ref[...] + b_ref[...]
    out_ref[...] = jnp.maximum(h, 0.0)


_tcmm = pl.pallas_call(
    _tcmm_body,
    grid=(GRID,),
    in_specs=[
        pl.BlockSpec((BN, D), lambda i: (i, 0)),
        pl.BlockSpec((D, D), lambda i: (0, 0)),
    ],
    out_specs=pl.BlockSpec((BN, D), lambda i: (i, 0)),
    out_shape=jax.ShapeDtypeStruct((N, D), jnp.float32),
)

_tc1 = pl.pallas_call(
    _tc1_body,
    grid=(GRID,),
    in_specs=[
        pl.BlockSpec((BN, D), lambda i: (i, 0)),
        pl.BlockSpec((NC, BN), lambda i: (0, i)),
    ],
    out_specs=(pl.BlockSpec((BN, D), lambda i: (i, 0)),
               pl.BlockSpec((BN, 1), lambda i: (i, 0))),
    out_shape=(jax.ShapeDtypeStruct((N, D), jnp.float32),
               jax.ShapeDtypeStruct((N, 1), jnp.float32)),
)

_tc2 = pl.pallas_call(
    _tc2_body,
    grid=(GRID,),
    in_specs=[
        pl.BlockSpec((NC, BN, D), lambda i: (0, i, 0)),
        pl.BlockSpec((BN, D), lambda i: (i, 0)),
        pl.BlockSpec((BN, 1), lambda i: (i, 0)),
        pl.BlockSpec((1, D), lambda i: (0, 0)),
        pl.BlockSpec((D, D), lambda i: (0, 0)),
    ],
    out_specs=pl.BlockSpec((BN, D), lambda i: (i, 0)),
    out_shape=jax.ShapeDtypeStruct((N, D), jnp.float32),
)

_tc3 = pl.pallas_call(
    _tc3_body,
    grid=(GRID,),
    in_specs=[
        pl.BlockSpec((NC, BN, D), lambda i: (0, i, 0)),
        pl.BlockSpec((BN, D), lambda i: (i, 0)),
        pl.BlockSpec((BN, 1), lambda i: (i, 0)),
        pl.BlockSpec((1, D), lambda i: (0, 0)),
    ],
    out_specs=pl.BlockSpec((BN, D), lambda i: (i, 0)),
    out_shape=jax.ShapeDtypeStruct((N, D), jnp.float32),
)


def kernel(x, edge_index, W1, b1, W2, b2):
    pad = EPAD - E
    pidx = jnp.arange(pad, dtype=jnp.int32)
    row_flat = jnp.concatenate([edge_index[0], pidx % MCHUNK])
    col_flat = jnp.concatenate([edge_index[1], N + pidx % (MPAD - N)])
    row_m = row_flat.reshape(NS, MTOT, MCHUNK)
    col_m = col_flat.reshape(NS, MTOT, MCHUNK)
    col_d = col_flat.reshape(NC, NS, NCHUNK, CHUNK)
    b1r = b1.reshape(1, D)
    b2r = b2.reshape(1, D)

    degp = _deg_kernel(col_d)
    xw1 = _tcmm(x, W1)
    y1, dinv = _tc1(xw1, degp)
    acc1 = _msg_kernel(y1, row_m, col_m)
    y2 = _tc2(acc1, y1, dinv, b1r, W2)
    acc2 = _msg_kernel(y2, row_m, col_m)
    return _tc3(acc2, y2, dinv, b2r)
